# baseline (speedup 1.0000x reference)
import functools

import jax
import jax.numpy as jnp
from jax import lax
from jax.experimental import pallas as pl
from jax.experimental.pallas import tpu as pltpu
from jax.experimental.pallas import tpu_sc as plsc

N_NODES = 10000
E_TOTAL = 320000
D_FEAT = 128
D_HID = 16
N_CLASSES = 10

NC, NS, LANES = 2, 16, 16
NW = NC * NS
CHUNK = 128
NCHUNK = 80
E_TILE = NCHUNK * CHUNK
N_PAD = 10240
DZ = N_PAD // NS
K = 5
NGROUP = NCHUNK // K
NSETS = 3
NB = 8
NR = N_NODES // NB
NRP = N_PAD * D_HID // 128

_mesh = plsc.VectorSubcoreMesh(core_axis_name="c", subcore_axis_name="s")


@functools.partial(
    pl.kernel,
    mesh=_mesh,
    compiler_params=pltpu.CompilerParams(use_tc_tiling_on_sc=False),
    out_type=jax.ShapeDtypeStruct((NC, N_PAD), jnp.float32),
    scratch_types=[
        pltpu.VMEM((NCHUNK, 2, CHUNK), jnp.int32),
        pltpu.VMEM((CHUNK,), jnp.float32),
        pltpu.VMEM((DZ,), jnp.float32),
        pltpu.VMEM_SHARED((N_PAD,), jnp.float32),
    ],
)
def _deg_kernel(ei_hbm, out_hbm, idx_v, ones_v, zb_v, acc_sh):
    c = lax.axis_index("c")
    s = lax.axis_index("s")
    wid = s * NC + c
    pltpu.sync_copy(ei_hbm.at[pl.ds(wid * NCHUNK, NCHUNK)], idx_v)
    one16 = jnp.ones((LANES,), jnp.float32)
    zero16 = jnp.zeros((LANES,), jnp.float32)
    for i in range(CHUNK // LANES):
        ones_v[pl.ds(i * LANES, LANES)] = one16
    for i in range(DZ // LANES):
        zb_v[pl.ds(i * LANES, LANES)] = zero16
    pltpu.sync_copy(zb_v, acc_sh.at[pl.ds(s * DZ, DZ)])
    plsc.subcore_barrier()

    def body(j, carry):
        pltpu.sync_copy(ones_v, acc_sh.at[idx_v.at[j, 1]], add=True)
        return carry

    lax.fori_loop(0, NCHUNK, body, 0)
    plsc.subcore_barrier()
    pltpu.sync_copy(acc_sh.at[pl.ds(s * DZ, DZ)],
                    out_hbm.at[c, pl.ds(s * DZ, DZ)])


@functools.partial(
    pl.kernel,
    mesh=_mesh,
    compiler_params=pltpu.CompilerParams(use_tc_tiling_on_sc=False),
    out_type=jax.ShapeDtypeStruct((NC, N_PAD, D_HID), jnp.float32),
    scratch_types=[
        pltpu.VMEM((NCHUNK, 2, CHUNK), jnp.int32),
        pltpu.VMEM((K * CHUNK, D_HID), jnp.float32),
        pltpu.VMEM((K * CHUNK, D_HID), jnp.float32),
        pltpu.VMEM((K * CHUNK, D_HID), jnp.float32),
        pltpu.VMEM((DZ, D_HID), jnp.float32),
        pltpu.VMEM_SHARED((N_PAD, D_HID), jnp.float32),
        pltpu.VMEM_SHARED((N_PAD, D_HID), jnp.float32),
        pltpu.SemaphoreType.DMA,
        pltpu.SemaphoreType.DMA,
        pltpu.SemaphoreType.DMA,
        pltpu.SemaphoreType.DMA,
        pltpu.SemaphoreType.DMA,
        pltpu.SemaphoreType.DMA,
    ],
)
def _agg_kernel(ei_hbm, g_hbm, out_hbm,
                eidx, buf0, buf1, buf2, zb, acc_sh, g_sh,
                gsem0, gsem1, gsem2, ssem0, ssem1, ssem2):
    c = lax.axis_index("c")
    s = lax.axis_index("s")
    wid = s * NC + c
    bufs = (buf0, buf1, buf2)
    gsems = (gsem0, gsem1, gsem2)
    ssems = (ssem0, ssem1, ssem2)

    pltpu.sync_copy(ei_hbm.at[pl.ds(wid * NCHUNK, NCHUNK)], eidx)
    zero16 = jnp.zeros((LANES,), jnp.float32)

    def zbody(i, carry):
        zb[i, :] = zero16
        return carry

    lax.fori_loop(0, DZ, zbody, 0)
    pltpu.sync_copy(zb, acc_sh.at[pl.ds(s * DZ, DZ)])
    pltpu.sync_copy(g_hbm.at[pl.ds(s * DZ, DZ)], g_sh.at[pl.ds(s * DZ, DZ)])
    plsc.subcore_barrier()

    def fire_gathers(g, st):
        for k in range(K):
            pltpu.async_copy(g_sh.at[eidx.at[g * K + k, 0]],
                             bufs[st].at[pl.ds(k * CHUNK, CHUNK)], gsems[st])

    def wait_gathers(st):
        for _ in range(K):
            pltpu.make_async_copy(g_sh.at[eidx.at[0, 0]],
                                  bufs[st].at[pl.ds(0, CHUNK)],
                                  gsems[st]).wait()

    def fire_scatters(g, st):
        for k in range(K):
            pltpu.async_copy(bufs[st].at[pl.ds(k * CHUNK, CHUNK)],
                             acc_sh.at[eidx.at[g * K + k, 1]], ssems[st],
                             add=True)

    def wait_scatters(st):
        for _ in range(K):
            pltpu.make_async_copy(bufs[st].at[pl.ds(0, CHUNK)],
                                  acc_sh.at[eidx.at[0, 1]], ssems[st]).wait()

    def part(g, st, drain_next, fire_next):
        if drain_next:
            wait_scatters((st + 1) % NSETS)
        if fire_next:
            fire_gathers(g + 1, (st + 1) % NSETS)
        wait_gathers(st)
        fire_scatters(g, st)

    fire_gathers(0, 0)
    part(0, 0, drain_next=False, fire_next=True)
    part(1, 1, drain_next=False, fire_next=True)
    part(2, 2, drain_next=True, fire_next=True)

    def gbody(t, carry):
        g = t * NSETS
        part(g + 0, 0, drain_next=True, fire_next=True)
        part(g + 1, 1, drain_next=True, fire_next=True)
        part(g + 2, 2, drain_next=True, fire_next=True)
        return carry

    lax.fori_loop(1, NGROUP // NSETS, gbody, 0)

    part(NGROUP - 1, 0, drain_next=True, fire_next=False)
    wait_scatters(2)
    wait_scatters(0)

    plsc.subcore_barrier()
    pltpu.sync_copy(acc_sh.at[pl.ds(s * DZ, DZ)],
                    out_hbm.at[c, pl.ds(s * DZ, DZ)])


def _t1_body(x_ref, w1_ref, d0_ref, d1_ref, g1_ref, dinv_ref):
    deg = d0_ref[...] + d1_ref[...] + 1.0
    dinv = lax.rsqrt(deg)
    w1 = w1_ref[...]
    hs = [jnp.dot(x_ref[pl.ds(NR * a, NR), :], w1,
                  preferred_element_type=jnp.float32) for a in range(NB)]
    h = jnp.concatenate(hs, axis=1)
    g1_ref[pl.ds(0, NR), :] = h * dinv
    dinv_ref[...] = dinv


_t1_call = pl.pallas_call(
    _t1_body,
    out_shape=[
        jax.ShapeDtypeStruct((NRP, 128), jnp.float32),
        jax.ShapeDtypeStruct((NR, 128), jnp.float32),
    ],
)


def _t2_body(a0_ref, a1_ref, g1_ref, dinv_ref, b1_ref, w2_ref, g2_ref):
    acc = a0_ref[...] + a1_ref[...]
    dinv = dinv_ref[...]
    z = jnp.maximum(dinv * (acc + g1_ref[pl.ds(0, NR), :]) + b1_ref[...], 0.0)
    h2 = jnp.dot(z, w2_ref[...], preferred_element_type=jnp.float32)
    g2_ref[pl.ds(0, NR), :] = h2 * dinv


_t2_call = pl.pallas_call(
    _t2_body,
    out_shape=jax.ShapeDtypeStruct((NRP, 128), jnp.float32),
)


def _t3_body(a0_ref, a1_ref, g2_ref, dinv_ref, b2_ref, summ_ref, out_ref):
    acc = a0_ref[...] + a1_ref[...]
    y = dinv_ref[...] * (acc + g2_ref[pl.ds(0, NR), :]) + b2_ref[...]
    lane = lax.broadcasted_iota(jnp.int32, (NR, 128), 1)
    mask = (lane % D_HID) < N_CLASSES
    z = jnp.where(mask, y, -1e30)
    m = jnp.max(z, axis=1, keepdims=True)
    e = jnp.where(mask, jnp.exp(z - m), 0.0)
    ssum = jnp.dot(e, summ_ref[...], preferred_element_type=jnp.float32)
    out_ref[...] = z - (m + jnp.log(ssum))


_t3_call = pl.pallas_call(
    _t3_body,
    out_shape=jax.ShapeDtypeStruct((NR, 128), jnp.float32),
)


def kernel(x, edge_index, W1, b1, W2, b2):
    ei = edge_index.astype(jnp.int32)
    eiT = NB * (ei % NR) + ei // NR
    npad = NW * E_TILE - E_TOTAL
    pad_src = jnp.zeros((1, npad), jnp.int32)
    pad_dst = (N_NODES
               + (jnp.arange(npad, dtype=jnp.int32) % CHUNK)).reshape(1, npad)
    eip = jnp.concatenate([eiT, jnp.concatenate([pad_src, pad_dst], 0)], 1)
    ei3 = eip.reshape(2, NW * NCHUNK, CHUNK).transpose(1, 0, 2)

    degx = _deg_kernel(ei3)
    d0 = jnp.broadcast_to(degx[0, :N_NODES].reshape(NR, NB, 1),
                          (NR, NB, D_HID)).reshape(NR, 128)
    d1 = jnp.broadcast_to(degx[1, :N_NODES].reshape(NR, NB, 1),
                          (NR, NB, D_HID)).reshape(NR, 128)
    b1p = jnp.tile(b1, NB).reshape(1, 128)
    w2p = jnp.pad(W2, ((0, 0), (0, D_HID - N_CLASSES)))
    w2bd = jnp.kron(jnp.eye(NB, dtype=jnp.float32), w2p)
    b2p = jnp.tile(jnp.pad(b2, (0, D_HID - N_CLASSES)), NB).reshape(1, 128)
    summ = jnp.kron(jnp.eye(NB, dtype=jnp.float32),
                    jnp.ones((D_HID, D_HID), jnp.float32))

    g1p, dinv = _t1_call(x, W1, d0, d1)
    accp1 = _agg_kernel(ei3, g1p.reshape(N_PAD, D_HID))
    a10 = accp1[0, :N_NODES, :].reshape(NR, 128)
    a11 = accp1[1, :N_NODES, :].reshape(NR, 128)
    g2p = _t2_call(a10, a11, g1p, dinv, b1p, w2bd)
    accp2 = _agg_kernel(ei3, g2p.reshape(N_PAD, D_HID))
    a20 = accp2[0, :N_NODES, :].reshape(NR, 128)
    a21 = accp2[1, :N_NODES, :].reshape(NR, 128)
    outp = _t3_call(a20, a21, g2p, dinv, b2p, summ)
    out = outp.reshape(NR, NB, D_HID)[:, :, :N_CLASSES]
    return out.transpose(1, 0, 2).reshape(N_NODES, N_CLASSES)


# speedup vs baseline: 1.3728x; 1.3728x over previous
import functools

import jax
import jax.numpy as jnp
from jax import lax
from jax.experimental import pallas as pl
from jax.experimental.pallas import tpu as pltpu
from jax.experimental.pallas import tpu_sc as plsc

N_NODES = 10000
E_TOTAL = 320000
D_FEAT = 128
D_HID = 16
N_CLASSES = 10

NC, NS, LANES = 2, 16, 16
NW = NC * NS
CHUNK = 128
NCHUNK = 80
E_TILE = NCHUNK * CHUNK
N_PAD = 10240
DZ = N_PAD // NS
K = 5
NGROUP = NCHUNK // K
NSETS = 3
NB = 8
NR = N_NODES // NB
NRP = N_PAD * D_HID // 128

_mesh = plsc.VectorSubcoreMesh(core_axis_name="c", subcore_axis_name="s")


@functools.partial(
    pl.kernel,
    mesh=_mesh,
    compiler_params=pltpu.CompilerParams(use_tc_tiling_on_sc=False),
    out_type=jax.ShapeDtypeStruct((NC, N_PAD), jnp.float32),
    scratch_types=[
        pltpu.VMEM((NCHUNK, 2, CHUNK), jnp.int32),
        pltpu.VMEM((CHUNK,), jnp.float32),
        pltpu.VMEM((DZ,), jnp.float32),
        pltpu.VMEM_SHARED((N_PAD,), jnp.float32),
    ],
)
def _deg_kernel(ei_hbm, out_hbm, idx_v, ones_v, zb_v, acc_sh):
    c = lax.axis_index("c")
    s = lax.axis_index("s")
    wid = s * NC + c
    pltpu.sync_copy(ei_hbm.at[pl.ds(wid * NCHUNK, NCHUNK)], idx_v)
    one16 = jnp.ones((LANES,), jnp.float32)
    zero16 = jnp.zeros((LANES,), jnp.float32)
    for i in range(CHUNK // LANES):
        ones_v[pl.ds(i * LANES, LANES)] = one16
    for i in range(DZ // LANES):
        zb_v[pl.ds(i * LANES, LANES)] = zero16
    pltpu.sync_copy(zb_v, acc_sh.at[pl.ds(s * DZ, DZ)])
    plsc.subcore_barrier()

    def body(j, carry):
        pltpu.sync_copy(ones_v, acc_sh.at[idx_v.at[j, 1]], add=True)
        return carry

    lax.fori_loop(0, NCHUNK, body, 0)
    plsc.subcore_barrier()
    pltpu.sync_copy(acc_sh.at[pl.ds(s * DZ, DZ)],
                    out_hbm.at[c, pl.ds(s * DZ, DZ)])


@functools.partial(
    pl.kernel,
    mesh=_mesh,
    compiler_params=pltpu.CompilerParams(use_tc_tiling_on_sc=False),
    out_type=jax.ShapeDtypeStruct((NC, N_PAD, D_HID), jnp.float32),
    scratch_types=[
        pltpu.VMEM((NCHUNK, 2, CHUNK), jnp.int32),
        pltpu.VMEM((K * CHUNK, D_HID), jnp.float32),
        pltpu.VMEM((K * CHUNK, D_HID), jnp.float32),
        pltpu.VMEM((K * CHUNK, D_HID), jnp.float32),
        pltpu.VMEM((DZ, D_HID), jnp.float32),
        pltpu.VMEM_SHARED((N_PAD, D_HID), jnp.float32),
        pltpu.VMEM_SHARED((N_PAD, D_HID), jnp.float32),
        pltpu.SemaphoreType.DMA,
        pltpu.SemaphoreType.DMA,
        pltpu.SemaphoreType.DMA,
        pltpu.SemaphoreType.DMA,
        pltpu.SemaphoreType.DMA,
        pltpu.SemaphoreType.DMA,
    ],
)
def _agg_kernel(ei_hbm, g_hbm, out_hbm,
                eidx, buf0, buf1, buf2, zb, acc_sh, g_sh,
                gsem0, gsem1, gsem2, ssem0, ssem1, ssem2):
    c = lax.axis_index("c")
    s = lax.axis_index("s")
    wid = s * NC + c
    bufs = (buf0, buf1, buf2)
    gsems = (gsem0, gsem1, gsem2)
    ssems = (ssem0, ssem1, ssem2)

    pltpu.sync_copy(ei_hbm.at[pl.ds(wid * NCHUNK, NCHUNK)], eidx)
    zero16 = jnp.zeros((LANES,), jnp.float32)

    def zbody(i, carry):
        zb[i, :] = zero16
        return carry

    lax.fori_loop(0, DZ, zbody, 0)
    pltpu.sync_copy(zb, acc_sh.at[pl.ds(s * DZ, DZ)])
    pltpu.sync_copy(g_hbm.at[pl.ds(s * DZ, DZ)], g_sh.at[pl.ds(s * DZ, DZ)])
    plsc.subcore_barrier()

    def fire_gathers(g, st):
        for k in range(K):
            pltpu.async_copy(g_sh.at[eidx.at[g * K + k, 0]],
                             bufs[st].at[pl.ds(k * CHUNK, CHUNK)], gsems[st])

    def wait_gathers(st):
        for _ in range(K):
            pltpu.make_async_copy(g_sh.at[eidx.at[0, 0]],
                                  bufs[st].at[pl.ds(0, CHUNK)],
                                  gsems[st]).wait()

    def fire_scatters(g, st):
        for k in range(K):
            pltpu.async_copy(bufs[st].at[pl.ds(k * CHUNK, CHUNK)],
                             acc_sh.at[eidx.at[g * K + k, 1]], ssems[st],
                             add=True)

    def wait_scatters(st):
        for _ in range(K):
            pltpu.make_async_copy(bufs[st].at[pl.ds(0, CHUNK)],
                                  acc_sh.at[eidx.at[0, 1]], ssems[st]).wait()

    def part(g, st, drain_next, fire_next):
        if drain_next:
            wait_scatters((st + 1) % NSETS)
        if fire_next:
            fire_gathers(g + 1, (st + 1) % NSETS)
        wait_gathers(st)
        fire_scatters(g, st)

    fire_gathers(0, 0)
    part(0, 0, drain_next=False, fire_next=True)
    part(1, 1, drain_next=False, fire_next=True)
    part(2, 2, drain_next=True, fire_next=True)

    def gbody(t, carry):
        g = t * NSETS
        part(g + 0, 0, drain_next=True, fire_next=True)
        part(g + 1, 1, drain_next=True, fire_next=True)
        part(g + 2, 2, drain_next=True, fire_next=True)
        return carry

    lax.fori_loop(1, NGROUP // NSETS, gbody, 0)

    part(NGROUP - 1, 0, drain_next=True, fire_next=False)
    wait_scatters(2)
    wait_scatters(0)

    plsc.subcore_barrier()
    pltpu.sync_copy(acc_sh.at[pl.ds(s * DZ, DZ)],
                    out_hbm.at[c, pl.ds(s * DZ, DZ)])


def _t1_body(x_ref, w1_ref, d0_ref, d1_ref, g1_ref, dinv_ref):
    deg = d0_ref[...] + d1_ref[...] + 1.0
    dinv = lax.rsqrt(deg)
    w1 = w1_ref[...]
    hs = [jnp.dot(x_ref[pl.ds(NR * a, NR), :], w1,
                  preferred_element_type=jnp.float32) for a in range(NB)]
    h = jnp.concatenate(hs, axis=1)
    g1_ref[pl.ds(0, NR), :] = h * dinv
    dinv_ref[...] = dinv


_t1_call = pl.pallas_call(
    _t1_body,
    out_shape=[
        jax.ShapeDtypeStruct((NRP, 128), jnp.float32),
        jax.ShapeDtypeStruct((NR, 128), jnp.float32),
    ],
)


def _t2_body(ap_ref, g1_ref, dinv_ref, b1_ref, w2_ref, g2_ref):
    acc = ap_ref[pl.ds(0, NR), :] + ap_ref[pl.ds(NRP, NR), :]
    dinv = dinv_ref[...]
    z = jnp.maximum(dinv * (acc + g1_ref[pl.ds(0, NR), :]) + b1_ref[...], 0.0)
    h2 = jnp.dot(z, w2_ref[...], preferred_element_type=jnp.float32)
    g2_ref[pl.ds(0, NR), :] = h2 * dinv


_t2_call = pl.pallas_call(
    _t2_body,
    out_shape=jax.ShapeDtypeStruct((NRP, 128), jnp.float32),
)


def _t3_body(ap_ref, g2_ref, dinv_ref, b2_ref, summ_ref, out_ref):
    acc = ap_ref[pl.ds(0, NR), :] + ap_ref[pl.ds(NRP, NR), :]
    y = dinv_ref[...] * (acc + g2_ref[pl.ds(0, NR), :]) + b2_ref[...]
    lane = lax.broadcasted_iota(jnp.int32, (NR, 128), 1)
    mask = (lane % D_HID) < N_CLASSES
    z = jnp.where(mask, y, -1e30)
    m = jnp.max(z, axis=1, keepdims=True)
    e = jnp.where(mask, jnp.exp(z - m), 0.0)
    ssum = jnp.dot(e, summ_ref[...], preferred_element_type=jnp.float32)
    out_ref[...] = z - (m + jnp.log(ssum))


_t3_call = pl.pallas_call(
    _t3_body,
    out_shape=jax.ShapeDtypeStruct((NR, 128), jnp.float32),
)


def kernel(x, edge_index, W1, b1, W2, b2):
    ei = edge_index.astype(jnp.int32)
    npad = NW * E_TILE - E_TOTAL
    pad_src = jnp.zeros((1, npad), jnp.int32)
    pad_dst = (N_NODES
               + (jnp.arange(npad, dtype=jnp.int32) % CHUNK)).reshape(1, npad)
    eip = jnp.concatenate([ei, jnp.concatenate([pad_src, pad_dst], 0)], 1)
    eip = jnp.where(eip < N_NODES, NB * (eip % NR) + eip // NR, eip)
    ei3 = eip.reshape(2, NW * NCHUNK, CHUNK).transpose(1, 0, 2)

    degx = _deg_kernel(ei3)
    d0 = jnp.broadcast_to(degx[0, :N_NODES].reshape(NR, NB, 1),
                          (NR, NB, D_HID)).reshape(NR, 128)
    d1 = jnp.broadcast_to(degx[1, :N_NODES].reshape(NR, NB, 1),
                          (NR, NB, D_HID)).reshape(NR, 128)
    b1p = jnp.tile(b1, NB).reshape(1, 128)
    w2p = jnp.pad(W2, ((0, 0), (0, D_HID - N_CLASSES)))
    w2bd = jnp.kron(jnp.eye(NB, dtype=jnp.float32), w2p)
    b2p = jnp.tile(jnp.pad(b2, (0, D_HID - N_CLASSES)), NB).reshape(1, 128)
    summ = jnp.kron(jnp.eye(NB, dtype=jnp.float32),
                    jnp.ones((D_HID, D_HID), jnp.float32))

    g1p, dinv = _t1_call(x, W1, d0, d1)
    accp1 = _agg_kernel(ei3, g1p.reshape(N_PAD, D_HID))
    g2p = _t2_call(accp1.reshape(NC * NRP, 128), g1p, dinv, b1p, w2bd)
    accp2 = _agg_kernel(ei3, g2p.reshape(N_PAD, D_HID))
    outp = _t3_call(accp2.reshape(NC * NRP, 128), g2p, dinv, b2p, summ)
    out = outp.reshape(NR, NB, D_HID)[:, :, :N_CLASSES]
    return out.transpose(1, 0, 2).reshape(N_NODES, N_CLASSES)
